# Initial kernel scaffold; baseline (speedup 1.0000x reference)
import jax
import jax.numpy as jnp
from jax.experimental import pallas as pl


def kernel(w, gamma, beta, seqs, seqL):
    raise NotImplementedError("write your pallas kernel here")


# speedup vs baseline: 1.6043x; 1.6043x over previous
import functools

import jax
import jax.numpy as jnp
from jax.experimental import pallas as pl
from jax.experimental.pallas import tpu as pltpu

_NEG_SLOPE = 0.01
_BN_EPS = 1e-5
_D_TILE = 8


def _conv_stats_kernel(xhw_ref, wk_ref, y_ref, stats_ref, *, d_tile, hw):
    mt = pl.program_id(1)
    ts = d_tile * hw
    acc = None
    for a in range(3):
        xs = xhw_ref[:, pl.ds((mt * d_tile + a) * hw, ts)]
        p = jax.lax.dot_general(
            xs, wk_ref[a],
            dimension_numbers=(((0,), (0,)), ((), ())),
            preferred_element_type=jnp.float32)
        acc = p if acc is None else acc + p
    yt = acc.T
    y_ref[...] = yt.astype(y_ref.dtype)
    stats_ref[:, 0:1] = jnp.sum(yt, axis=-1, keepdims=True)
    stats_ref[:, 1:2] = jnp.sum(yt * yt, axis=-1, keepdims=True)


def _bn_act_kernel(y_ref, scale_ref, shift_ref, o_ref):
    y = y_ref[...].astype(jnp.float32) * scale_ref[...] + shift_ref[...]
    o_ref[...] = jnp.where(y >= 0.0, y, _NEG_SLOPE * y)


def kernel(w, gamma, beta, seqs, seqL):
    del seqL
    N, C, D, H, W = seqs.shape
    Cout = w.shape[0]
    hw = H * W
    ms = D * hw
    n_tiles = D // _D_TILE
    ts = _D_TILE * hw

    xb = seqs.astype(jnp.bfloat16)
    xq = jnp.pad(xb, ((0, 0), (0, 0), (1, 1), (1, 1), (1, 1)))
    taps = [xq[:, :, :, b:b + H, c:c + W] for b in range(3) for c in range(3)]
    xhw = jnp.stack(taps, axis=1)
    xhw = xhw.reshape(N, 9 * C, (D + 2) * hw)

    wk = w.transpose(2, 3, 4, 1, 0).reshape(3, 9 * C, Cout).astype(jnp.bfloat16)

    cost1 = pl.CostEstimate(
        flops=2 * N * ms * (27 * C) * Cout,
        transcendentals=0,
        bytes_accessed=(N * 9 * C * (D + 2) * hw * 2 + 3 * 9 * C * Cout * 2
                       + N * Cout * ms * 2 + N * n_tiles * Cout * 2 * 4))

    y_cm, part_stats = pl.pallas_call(
        functools.partial(_conv_stats_kernel, d_tile=_D_TILE, hw=hw),
        out_shape=(jax.ShapeDtypeStruct((N, Cout, ms), jnp.bfloat16),
                   jax.ShapeDtypeStruct((N, n_tiles, Cout, 2), jnp.float32)),
        grid=(N, n_tiles),
        in_specs=[pl.BlockSpec((None, 9 * C, (D + 2) * hw), lambda n, i: (n, 0, 0)),
                  pl.BlockSpec((3, 9 * C, Cout), lambda n, i: (0, 0, 0))],
        out_specs=[pl.BlockSpec((None, Cout, ts), lambda n, i: (n, 0, i)),
                   pl.BlockSpec((None, None, Cout, 2), lambda n, i: (n, i, 0, 0))],
        compiler_params=pltpu.CompilerParams(
            dimension_semantics=("parallel", "arbitrary")),
        cost_estimate=cost1,
    )(xhw, wk)

    M = N * ms
    stats = jnp.sum(part_stats, axis=(0, 1))
    mean = stats[:, 0] / M
    var = jnp.maximum(stats[:, 1] / M - mean * mean, 0.0)
    scale = gamma / jnp.sqrt(var + _BN_EPS)
    shift = beta - mean * scale
    scale_c = scale.reshape(Cout, 1).astype(jnp.float32)
    shift_c = shift.reshape(Cout, 1).astype(jnp.float32)

    cost2 = pl.CostEstimate(
        flops=4 * N * ms * Cout,
        transcendentals=0,
        bytes_accessed=N * ms * Cout * (2 + 4) + 2 * Cout * 4)

    out_cm = pl.pallas_call(
        _bn_act_kernel,
        out_shape=jax.ShapeDtypeStruct((N, Cout, ms), jnp.float32),
        grid=(N, n_tiles),
        in_specs=[pl.BlockSpec((None, Cout, ts), lambda n, i: (n, 0, i)),
                  pl.BlockSpec((Cout, 1), lambda n, i: (0, 0)),
                  pl.BlockSpec((Cout, 1), lambda n, i: (0, 0))],
        out_specs=pl.BlockSpec((None, Cout, ts), lambda n, i: (n, 0, i)),
        compiler_params=pltpu.CompilerParams(
            dimension_semantics=("parallel", "parallel")),
        cost_estimate=cost2,
    )(y_cm, scale_c, shift_c)

    return out_cm.reshape(N, Cout, D, H, W)
